# baseline (speedup 1.0000x reference)
import functools

import jax
import jax.numpy as jnp
from jax import lax
from jax.experimental import pallas as pl
from jax.experimental.pallas import tpu as pltpu
from jax.experimental.pallas import tpu_sc as plsc

EMB = 64
S = 128


@functools.lru_cache(maxsize=None)
def _build_sc_kernel(B, NT, C, NW):
    RPW = B // NW
    NCHUNKS = RPW // C
    NB = C // S
    assert B == RPW * NW and RPW == NCHUNKS * C and C == NB * S
    assert NCHUNKS >= 4 and NCHUNKS % 2 == 0

    mesh = plsc.VectorSubcoreMesh(core_axis_name="c", subcore_axis_name="s")

    @functools.partial(
        pl.kernel,
        mesh=mesh,
        compiler_params=pltpu.CompilerParams(
            use_tc_tiling_on_sc=False, needs_layout_passes=False),
        out_type=jax.ShapeDtypeStruct((B, EMB), jnp.float32),
        scratch_types=[
            [pltpu.VMEM((C,), jnp.int32) for _ in range(2)],
            [pltpu.VMEM((C, EMB), jnp.float32) for _ in range(2)],
            [pltpu.SemaphoreType.DMA for _ in range(2)],
            [pltpu.SemaphoreType.DMA for _ in range(2)],
            [pltpu.SemaphoreType.DMA for _ in range(2)],
        ],
    )
    def k(ids_hbm, table_hbm, out_hbm, ids_v, buf, isem, gsem, osem):
        wid = lax.axis_index("s") * 2 + lax.axis_index("c")
        tile_base = wid * RPW

        def ids_copy(c, p):
            return pltpu.make_async_copy(
                ids_hbm.at[pl.ds(tile_base + c * C, C)], ids_v[p], isem[p])

        ids_copy(0, 0).start()
        ids_copy(1, 1).start()

        def half_chunk(i, p):
            c = 2 * i + p
            base = tile_base + c * C

            @pl.when(c >= 2)
            def _():
                pltpu.make_async_copy(
                    buf[p], out_hbm.at[pl.ds(base, C)], osem[p]).wait()

            ids_copy(c, p).wait()

            for kb in range(NB):
                pltpu.make_async_copy(
                    table_hbm.at[ids_v[p].at[pl.ds(kb * S, S)]],
                    buf[p].at[pl.ds(kb * S, S)], gsem[p]).start()

            for kb in range(NB):
                pltpu.make_async_copy(
                    table_hbm.at[ids_v[p].at[pl.ds(kb * S, S)]],
                    buf[p].at[pl.ds(kb * S, S)], gsem[p]).wait()

            @pl.when(c + 2 < NCHUNKS)
            def _():
                ids_copy(c + 2, p).start()

            pltpu.make_async_copy(
                buf[p], out_hbm.at[pl.ds(base, C)], osem[p]).start()
            return 0

        def pair_body(i, x):
            half_chunk(i, 0)
            half_chunk(i, 1)
            return x

        lax.fori_loop(0, NCHUNKS // 2, pair_body, 0)

        for p in range(2):
            c = NCHUNKS - 2 + p
            pltpu.make_async_copy(
                buf[p], out_hbm.at[pl.ds(tile_base + c * C, C)], osem[p]).wait()

    return k


def kernel(node_ids, user_table, item_table):
    nb, nn = node_ids.shape
    B = nb * nn
    ids = node_ids.reshape(B).astype(jnp.int32)
    table = jnp.concatenate(
        [jnp.zeros((1, EMB), jnp.float32),
         user_table.astype(jnp.float32),
         item_table.astype(jnp.float32)], axis=0)
    k = _build_sc_kernel(B, int(table.shape[0]), C=640, NW=32)
    out = k(ids, table)
    return out.reshape(nb, nn, EMB)
